# Initial kernel scaffold; baseline (speedup 1.0000x reference)
import jax
import jax.numpy as jnp
from jax.experimental import pallas as pl


def kernel(x, w1, b1, w2, b2, w_out_t, b_out):
    raise NotImplementedError("write your pallas kernel here")


# speedup vs baseline: 2.2050x; 2.2050x over previous
import jax
import jax.numpy as jnp
from jax.experimental import pallas as pl
from jax.experimental.pallas import tpu as pltpu

_G = 32
_N = _G * _G
_K = 5
_HALF = _K // 2
_TAPS = _K * _K
_C1 = 16
_C2 = 32
_NB = 16


def _shifted(v, off):
    n = v.shape[-1]
    off = off % n
    if off == 0:
        return v
    return pltpu.roll(v, n - off, v.ndim - 1)


def _cnn_block_kernel(x_ref, w1_ref, b1_ref, w2_ref, b2_ref, o_ref,
                      p1_ref, p2_ref, a1_ref):
    n = _N
    x = x_ref[...]
    for t in range(_TAPS):
        di, dj = divmod(t, _K)
        off = (di - _HALF) * _G + (dj - _HALF)
        p1_ref[:, t, :] = _shifted(x, off).astype(jnp.bfloat16)

    w1 = w1_ref[...]
    b1 = b1_ref[...]
    for i in range(_NB):
        a = jnp.dot(w1, p1_ref[i], preferred_element_type=jnp.float32)
        a1_ref[i] = jnp.maximum(a + b1, 0.0)

    a1 = a1_ref[...].reshape(_NB * _C1, n)
    m1 = jnp.maximum(jnp.maximum(a1, _shifted(a1, 1)),
                     jnp.maximum(_shifted(a1, _G), _shifted(a1, _G + 1)))
    lane = jax.lax.broadcasted_iota(jnp.int32, (1, n), 1)
    row, col = lane // _G, lane % _G
    ok = ((row % 2 == 0) & (row >= 2) & (row <= _G - 4) &
          (col % 2 == 0) & (col >= 2) & (col <= _G - 4))
    m1 = jnp.where(ok, m1, 0.0)

    for djx in range(_K):
        cd = _shifted(m1, 2 * (djx - _HALF))
        for dix in range(_K):
            t = dix * _K + djx
            slab = _shifted(cd, 2 * (dix - _HALF) * _G)
            p2_ref[:, t * _C1:(t + 1) * _C1, :] = (
                slab.astype(jnp.bfloat16).reshape(_NB, _C1, n))

    w2 = w2_ref[...]
    b2 = b2_ref[...]
    for i in range(_NB):
        a = jnp.dot(w2, p2_ref[i], preferred_element_type=jnp.float32)
        o_ref[i] = jnp.maximum(a + b2, 0.0)

    a2 = o_ref[...].reshape(_NB * _C2, n)
    m2 = jnp.maximum(jnp.maximum(a2, _shifted(a2, 2)),
                     jnp.maximum(_shifted(a2, 2 * _G), _shifted(a2, 2 * _G + 2)))
    o_ref[...] = m2.reshape(_NB, _C2, n)


def kernel(x, w1, b1, w2, b2, w_out_t, b_out):
    b = x.shape[0]
    xg = jnp.pad(x, ((0, 0), (0, 0), (2, 2), (2, 2)))
    xg = xg.reshape(b, _N).astype(jnp.float32)

    w1m = jnp.transpose(w1[:, :, 0]).astype(jnp.bfloat16)
    w2m = jnp.transpose(w2, (1, 0, 2)).reshape(_C2, _TAPS * _C1)
    w2m = w2m.astype(jnp.bfloat16)

    feat_dilated = pl.pallas_call(
        _cnn_block_kernel,
        out_shape=jax.ShapeDtypeStruct((b, _C2, _N), jnp.float32),
        grid=(b // _NB,),
        in_specs=[
            pl.BlockSpec((_NB, _N), lambda i: (i, 0)),
            pl.BlockSpec((_C1, _TAPS), lambda i: (0, 0)),
            pl.BlockSpec((_C1, 1), lambda i: (0, 0)),
            pl.BlockSpec((_C2, _TAPS * _C1), lambda i: (0, 0)),
            pl.BlockSpec((_C2, 1), lambda i: (0, 0)),
        ],
        out_specs=pl.BlockSpec((_NB, _C2, _N), lambda i: (i, 0, 0)),
        scratch_shapes=[
            pltpu.VMEM((_NB, _TAPS, _N), jnp.bfloat16),
            pltpu.VMEM((_NB, _TAPS * _C1, _N), jnp.bfloat16),
            pltpu.VMEM((_NB, _C1, _N), jnp.float32),
        ],
        compiler_params=pltpu.CompilerParams(
            dimension_semantics=("parallel",)),
    )(xg, w1m, b1, w2m, b2)

    hw = 7
    feat = feat_dilated.reshape(b, _C2, _G, _G)[:, :, 2:2 + 4 * hw:4,
                                                2:2 + 4 * hw:4]
    feat = feat.reshape(b, _C2 * hw * hw)
    logits = jnp.dot(feat, w_out_t,
                     precision=jax.lax.Precision.HIGHEST) + b_out
    return logits, feat


# speedup vs baseline: 5.1694x; 2.3444x over previous
import jax
import jax.numpy as jnp
from jax.experimental import pallas as pl
from jax.experimental.pallas import tpu as pltpu

_G = 32
_N = _G * _G
_K = 5
_HALF = _K // 2
_TAPS = _K * _K
_C1 = 16
_C2 = 32
_NB = 16


def _shifted(v, off):
    n = v.shape[-1]
    off = off % n
    if off == 0:
        return v
    return pltpu.roll(v, n - off, v.ndim - 1)


def _shifted_b(v, off):
    n = v.shape[-1]
    off = off % n
    if off == 0:
        return v
    vi = pltpu.bitcast(v, jnp.int32)
    vi = pltpu.roll(vi, n - off, vi.ndim - 1)
    return pltpu.bitcast(vi, jnp.bfloat16)


def _cnn_block_kernel(x_ref, w1_ref, b1_ref, w2_ref, b2_ref, o_ref,
                      p1_ref, p2_ref, a1_ref):
    n = _N
    x = x_ref[...]
    for t in range(_TAPS):
        di, dj = divmod(t, _K)
        off = (di - _HALF) * _G + (dj - _HALF)
        p1_ref[:, t, :] = _shifted(x, off).astype(jnp.bfloat16)

    w1 = w1_ref[...]
    b1 = b1_ref[...]
    for i in range(_NB):
        a = jnp.dot(w1, p1_ref[i], preferred_element_type=jnp.float32)
        a1_ref[i] = jnp.maximum(a + b1, 0.0)

    a1 = a1_ref[...].reshape(_NB * _C1, n).astype(jnp.bfloat16)
    m1 = jnp.maximum(jnp.maximum(a1, _shifted_b(a1, 1)),
                     jnp.maximum(_shifted_b(a1, _G), _shifted_b(a1, _G + 1)))
    lane = jax.lax.broadcasted_iota(jnp.int32, (1, n), 1)
    row, col = lane // _G, lane % _G
    ok = ((row % 2 == 0) & (row >= 2) & (row <= _G - 4) &
          (col % 2 == 0) & (col >= 2) & (col <= _G - 4))
    m1 = jnp.where(ok, m1, jnp.bfloat16(0.0))

    for djx in range(_K):
        cd = _shifted_b(m1, 2 * (djx - _HALF))
        for dix in range(_K):
            t = dix * _K + djx
            slab = _shifted_b(cd, 2 * (dix - _HALF) * _G)
            p2_ref[:, t * _C1:(t + 1) * _C1, :] = slab.reshape(_NB, _C1, n)

    w2 = w2_ref[...]
    b2 = b2_ref[...]
    for i in range(_NB):
        a = jnp.dot(w2, p2_ref[i], preferred_element_type=jnp.float32)
        o_ref[i] = jnp.maximum(a + b2, 0.0).astype(jnp.bfloat16)

    a2 = o_ref[...].reshape(_NB * _C2, n)
    m2 = jnp.maximum(jnp.maximum(a2, _shifted_b(a2, 2)),
                     jnp.maximum(_shifted_b(a2, 2 * _G), _shifted_b(a2, 2 * _G + 2)))
    o_ref[...] = m2.reshape(_NB, _C2, n)


def kernel(x, w1, b1, w2, b2, w_out_t, b_out):
    b = x.shape[0]
    xg = jnp.pad(x, ((0, 0), (0, 0), (2, 2), (2, 2)))
    xg = xg.reshape(b, _N).astype(jnp.float32)

    w1m = jnp.transpose(w1[:, :, 0]).astype(jnp.bfloat16)
    w2m = jnp.transpose(w2, (1, 0, 2)).reshape(_C2, _TAPS * _C1)
    w2m = w2m.astype(jnp.bfloat16)

    feat_dilated = pl.pallas_call(
        _cnn_block_kernel,
        out_shape=jax.ShapeDtypeStruct((b, _C2, _N), jnp.bfloat16),
        grid=(b // _NB,),
        in_specs=[
            pl.BlockSpec((_NB, _N), lambda i: (i, 0)),
            pl.BlockSpec((_C1, _TAPS), lambda i: (0, 0)),
            pl.BlockSpec((_C1, 1), lambda i: (0, 0)),
            pl.BlockSpec((_C2, _TAPS * _C1), lambda i: (0, 0)),
            pl.BlockSpec((_C2, 1), lambda i: (0, 0)),
        ],
        out_specs=pl.BlockSpec((_NB, _C2, _N), lambda i: (i, 0, 0)),
        scratch_shapes=[
            pltpu.VMEM((_NB, _TAPS, _N), jnp.bfloat16),
            pltpu.VMEM((_NB, _TAPS * _C1, _N), jnp.bfloat16),
            pltpu.VMEM((_NB, _C1, _N), jnp.float32),
        ],
        compiler_params=pltpu.CompilerParams(
            dimension_semantics=("parallel",)),
    )(xg, w1m, b1, w2m, b2)

    hw = 7
    feat = feat_dilated.reshape(b, _C2, _G, _G)[:, :, 2:2 + 4 * hw:4,
                                                2:2 + 4 * hw:4]
    feat = feat.reshape(b, _C2 * hw * hw).astype(jnp.float32)
    logits = jnp.dot(feat, w_out_t,
                     precision=jax.lax.Precision.HIGHEST) + b_out
    return logits, feat


# speedup vs baseline: 5.8425x; 1.1302x over previous
import numpy as np
import jax
import jax.numpy as jnp
from jax.experimental import pallas as pl
from jax.experimental.pallas import tpu as pltpu

_G = 32
_N = _G * _G
_K = 5
_HALF = _K // 2
_TAPS = _K * _K
_C1 = 16
_C2 = 32
_NB = 32
_H2 = 18
_W2 = 21
_N2 = 384


def _sel_matrix():
    s = np.zeros((_N, _N2), np.float32)
    for e in range(14):
        for f in range(14):
            s[(2 + 2 * e) * _G + 2 + 2 * f, (2 + e) * _W2 + 2 + f] = 1.0
    return s


def _shifted(v, off):
    n = v.shape[-1]
    off = off % n
    if off == 0:
        return v
    return pltpu.roll(v, n - off, v.ndim - 1)


def _shifted_b(v, off):
    n = v.shape[-1]
    off = off % n
    if off == 0:
        return v
    vi = pltpu.bitcast(v, jnp.int32)
    vi = pltpu.roll(vi, n - off, vi.ndim - 1)
    return pltpu.bitcast(vi, jnp.bfloat16)


def _cnn_block_kernel(x_ref, w1_ref, b1_ref, w2_ref, b2_ref, s_ref, o_ref,
                      p1_ref, p2_ref, a1_ref):
    n = _N
    x = x_ref[...]
    for t in range(_TAPS):
        di, dj = divmod(t, _K)
        off = (di - _HALF) * _G + (dj - _HALF)
        p1_ref[:, t, :] = _shifted(x, off)

    w1 = w1_ref[...]
    b1 = b1_ref[...]
    for i in range(_NB):
        a = jnp.dot(w1, p1_ref[i], preferred_element_type=jnp.float32)
        a1_ref[i] = jnp.maximum(a + b1, 0.0).astype(jnp.bfloat16)

    a1 = a1_ref[...].reshape(_NB * _C1, n)
    m1 = jnp.maximum(a1, _shifted_b(a1, 1))
    m1 = jnp.maximum(m1, _shifted_b(m1, _G))

    mc = jnp.dot(m1, s_ref[...],
                 preferred_element_type=jnp.float32).astype(jnp.bfloat16)

    w2 = w2_ref[...]
    b2 = b2_ref[...]
    for g in range(0, _NB, 2):
        mg = mc[g * _C1:(g + 2) * _C1]
        for dix in range(_K):
            for djx in range(_K):
                t = dix * _K + djx
                off = (dix - _HALF) * _W2 + (djx - _HALF)
                slab = _shifted_b(mg, off)
                p2_ref[g:g + 2, t * _C1:(t + 1) * _C1, :] = (
                    slab.reshape(2, _C1, _N2))
        for i in (g, g + 1):
            a = jnp.dot(w2, p2_ref[i], preferred_element_type=jnp.float32)
            o_ref[i] = jnp.maximum(a + b2, 0.0).astype(jnp.bfloat16)

    a2 = o_ref[...].reshape(_NB * _C2, _N2)
    m2 = jnp.maximum(a2, _shifted_b(a2, 1))
    m2 = jnp.maximum(m2, _shifted_b(m2, _W2))
    o_ref[...] = m2.reshape(_NB, _C2, _N2)


def kernel(x, w1, b1, w2, b2, w_out_t, b_out):
    b = x.shape[0]
    xg = jnp.pad(x, ((0, 0), (0, 0), (2, 2), (2, 2)))
    xg = xg.reshape(b, _N).astype(jnp.float32)

    w1m = jnp.transpose(w1[:, :, 0]).astype(jnp.bfloat16)
    w2m = jnp.transpose(w2, (1, 0, 2)).reshape(_C2, _TAPS * _C1)
    w2m = w2m.astype(jnp.bfloat16)
    sel = jnp.asarray(_sel_matrix(), jnp.bfloat16)

    feat_c = pl.pallas_call(
        _cnn_block_kernel,
        out_shape=jax.ShapeDtypeStruct((b, _C2, _N2), jnp.bfloat16),
        grid=(b // _NB,),
        in_specs=[
            pl.BlockSpec((_NB, _N), lambda i: (i, 0)),
            pl.BlockSpec((_C1, _TAPS), lambda i: (0, 0)),
            pl.BlockSpec((_C1, 1), lambda i: (0, 0)),
            pl.BlockSpec((_C2, _TAPS * _C1), lambda i: (0, 0)),
            pl.BlockSpec((_C2, 1), lambda i: (0, 0)),
            pl.BlockSpec((_N, _N2), lambda i: (0, 0)),
        ],
        out_specs=pl.BlockSpec((_NB, _C2, _N2), lambda i: (i, 0, 0)),
        scratch_shapes=[
            pltpu.VMEM((_NB, _TAPS, _N), jnp.float32),
            pltpu.VMEM((_NB, _TAPS * _C1, _N2), jnp.bfloat16),
            pltpu.VMEM((_NB, _C1, _N), jnp.bfloat16),
        ],
        compiler_params=pltpu.CompilerParams(
            dimension_semantics=("parallel",)),
    )(xg, w1m, b1, w2m, b2, sel)

    hw = 7
    feat = feat_c[:, :, :_H2 * _W2].reshape(b, _C2, _H2, _W2)
    feat = feat[:, :, 2:2 + 2 * hw:2, 2:2 + 2 * hw:2]
    feat = feat.reshape(b, _C2 * hw * hw).astype(jnp.float32)
    logits = jnp.dot(feat, w_out_t,
                     precision=jax.lax.Precision.HIGHEST) + b_out
    return logits, feat


# speedup vs baseline: 6.0523x; 1.0359x over previous
import numpy as np
import jax
import jax.numpy as jnp
from jax.experimental import pallas as pl
from jax.experimental.pallas import tpu as pltpu

_G = 32
_N = _G * _G
_K = 5
_HALF = _K // 2
_TAPS = _K * _K
_C1 = 16
_C2 = 32
_NB = 64
_H2 = 18
_W2 = 21
_N2 = 384


def _sel_matrix():
    s = np.zeros((_N, _N2), np.float32)
    for e in range(14):
        for f in range(14):
            s[(2 + 2 * e) * _G + 2 + 2 * f, (2 + e) * _W2 + 2 + f] = 1.0
    return s


def _shifted(v, off):
    n = v.shape[-1]
    off = off % n
    if off == 0:
        return v
    return pltpu.roll(v, n - off, v.ndim - 1)


def _shifted_b(v, off):
    n = v.shape[-1]
    off = off % n
    if off == 0:
        return v
    vi = pltpu.bitcast(v, jnp.int32)
    vi = pltpu.roll(vi, n - off, vi.ndim - 1)
    return pltpu.bitcast(vi, jnp.bfloat16)


def _cnn_block_kernel(x_ref, w1_ref, b1_ref, w2_ref, b2_ref, s_ref, o_ref,
                      p1_ref, p2_ref, a1_ref):
    n = _N
    x = x_ref[...]
    for t in range(_TAPS):
        di, dj = divmod(t, _K)
        off = (di - _HALF) * _G + (dj - _HALF)
        p1_ref[:, t, :] = _shifted(x, off)

    w1 = w1_ref[...]
    b1 = b1_ref[...]
    for i in range(_NB):
        a = jnp.dot(w1, p1_ref[i], preferred_element_type=jnp.float32)
        a1_ref[i] = jnp.maximum(a + b1, 0.0).astype(jnp.bfloat16)

    a1 = a1_ref[...].reshape(_NB * _C1, n)
    m1 = jnp.maximum(a1, _shifted_b(a1, 1))
    m1 = jnp.maximum(m1, _shifted_b(m1, _G))

    mc = jnp.dot(m1, s_ref[...],
                 preferred_element_type=jnp.float32).astype(jnp.bfloat16)

    w2 = w2_ref[...]
    b2 = b2_ref[...]
    for g in range(0, _NB, 2):
        mg = mc[g * _C1:(g + 2) * _C1]
        for dix in range(_K):
            for djx in range(_K):
                t = dix * _K + djx
                off = (dix - _HALF) * _W2 + (djx - _HALF)
                slab = _shifted_b(mg, off)
                p2_ref[g:g + 2, t * _C1:(t + 1) * _C1, :] = (
                    slab.reshape(2, _C1, _N2))
        for i in (g, g + 1):
            a = jnp.dot(w2, p2_ref[i], preferred_element_type=jnp.float32)
            o_ref[i] = jnp.maximum(a + b2, 0.0).astype(jnp.bfloat16)

    a2 = o_ref[...].reshape(_NB * _C2, _N2)
    m2 = jnp.maximum(a2, _shifted_b(a2, 1))
    m2 = jnp.maximum(m2, _shifted_b(m2, _W2))
    o_ref[...] = m2.reshape(_NB, _C2, _N2)


def kernel(x, w1, b1, w2, b2, w_out_t, b_out):
    b = x.shape[0]
    xg = jnp.pad(x, ((0, 0), (0, 0), (2, 2), (2, 2)))
    xg = xg.reshape(b, _N).astype(jnp.float32)

    w1m = jnp.transpose(w1[:, :, 0]).astype(jnp.bfloat16)
    w2m = jnp.transpose(w2, (1, 0, 2)).reshape(_C2, _TAPS * _C1)
    w2m = w2m.astype(jnp.bfloat16)
    sel = jnp.asarray(_sel_matrix(), jnp.bfloat16)

    feat_c = pl.pallas_call(
        _cnn_block_kernel,
        out_shape=jax.ShapeDtypeStruct((b, _C2, _N2), jnp.bfloat16),
        grid=(b // _NB,),
        in_specs=[
            pl.BlockSpec((_NB, _N), lambda i: (i, 0)),
            pl.BlockSpec((_C1, _TAPS), lambda i: (0, 0)),
            pl.BlockSpec((_C1, 1), lambda i: (0, 0)),
            pl.BlockSpec((_C2, _TAPS * _C1), lambda i: (0, 0)),
            pl.BlockSpec((_C2, 1), lambda i: (0, 0)),
            pl.BlockSpec((_N, _N2), lambda i: (0, 0)),
        ],
        out_specs=pl.BlockSpec((_NB, _C2, _N2), lambda i: (i, 0, 0)),
        scratch_shapes=[
            pltpu.VMEM((_NB, _TAPS, _N), jnp.float32),
            pltpu.VMEM((_NB, _TAPS * _C1, _N2), jnp.bfloat16),
            pltpu.VMEM((_NB, _C1, _N), jnp.bfloat16),
        ],
        compiler_params=pltpu.CompilerParams(
            dimension_semantics=("parallel",)),
    )(xg, w1m, b1, w2m, b2, sel)

    hw = 7
    feat = feat_c[:, :, :_H2 * _W2].reshape(b, _C2, _H2, _W2)
    feat = feat[:, :, 2:2 + 2 * hw:2, 2:2 + 2 * hw:2]
    feat = feat.reshape(b, _C2 * hw * hw).astype(jnp.float32)
    logits = jnp.dot(feat, w_out_t,
                     precision=jax.lax.Precision.HIGHEST) + b_out
    return logits, feat
